# baseline (speedup 1.0000x reference)
import jax
import jax.numpy as jnp
from jax import lax
from jax.experimental import pallas as pl
from jax.experimental.pallas import tpu as pltpu
from jax.experimental.pallas import tpu_sc as plsc

NC = 2
NS = 16
NW = NC * NS
LANES = 16

BATCH = 4
SEQ_LEN = 4096
D_MODEL = 2048
ROWS_PER_W = SEQ_LEN // NW
TILE_R = 4
TILE_E = TILE_R * D_MODEL
N_CHUNKS = ROWS_PER_W // TILE_R


def _sc_body(x_hbm, pos_hbm, out_hbm, *scratch):
    pos_v = scratch[0:2]
    x_v = (scratch[2:6], scratch[6:10])
    ld = scratch[10:12]
    st = scratch[12:14]

    wid = lax.axis_index("s") * NC + lax.axis_index("c")
    e0 = wid * ROWS_PER_W * D_MODEL

    def pos_off(k):
        return e0 + k * TILE_E

    def x_off(k, b):
        return b * SEQ_LEN * D_MODEL + pos_off(k)

    def start_loads(k, ring):
        pltpu.async_copy(pos_hbm.at[pl.ds(pos_off(k), TILE_E)], pos_v[ring], ld[ring])
        for b in range(BATCH):
            pltpu.async_copy(
                x_hbm.at[pl.ds(x_off(k, b), TILE_E)], x_v[ring][b], ld[ring]
            )

    def wait_loads(k, ring):
        pltpu.make_async_copy(
            pos_hbm.at[pl.ds(pos_off(k), TILE_E)], pos_v[ring], ld[ring]
        ).wait()
        for b in range(BATCH):
            pltpu.make_async_copy(
                x_hbm.at[pl.ds(x_off(k, b), TILE_E)], x_v[ring][b], ld[ring]
            ).wait()

    def start_stores(k, ring):
        for b in range(BATCH):
            pltpu.async_copy(
                x_v[ring][b], out_hbm.at[pl.ds(x_off(k, b), TILE_E)], st[ring]
            )

    def wait_stores(k, ring):
        for b in range(BATCH):
            pltpu.make_async_copy(
                x_v[ring][b], out_hbm.at[pl.ds(x_off(k, b), TILE_E)], st[ring]
            ).wait()

    def compute(ring):
        bufs = x_v[ring]
        pv = pos_v[ring]

        @plsc.parallel_loop(0, TILE_E, step=LANES, unroll=16)
        def _(j):
            p = pv[pl.ds(j, LANES)]
            for b in range(BATCH):
                bufs[b][pl.ds(j, LANES)] = bufs[b][pl.ds(j, LANES)] + p

    start_loads(0, 0)
    start_loads(1, 1)

    def pair_body(p, _):
        k0 = p * 2
        for ring in range(2):
            k = k0 + ring
            wait_loads(k, ring)
            compute(ring)
            start_stores(k, ring)

        for ring in range(2):
            k = k0 + ring

            @pl.when(k + 2 < N_CHUNKS)
            def _():
                wait_stores(k, ring)
                start_loads(k + 2, ring)

        return 0

    lax.fori_loop(0, N_CHUNKS // 2, pair_body, 0)

    wait_stores(N_CHUNKS - 2, 0)
    wait_stores(N_CHUNKS - 1, 1)


def _sc_add(x1, pos1):
    k = pl.kernel(
        _sc_body,
        out_type=jax.ShapeDtypeStruct((BATCH * SEQ_LEN * D_MODEL,), jnp.float32),
        mesh=plsc.VectorSubcoreMesh(core_axis_name="c", subcore_axis_name="s"),
        scratch_types=(
            [pltpu.VMEM((TILE_E,), jnp.float32) for _ in range(10)]
            + [pltpu.SemaphoreType.DMA for _ in range(4)]
        ),
    )
    return k(x1, pos1)


def kernel(x, pos_table):
    batch, seq_len, d_model = x.shape
    x1 = x.reshape(batch * seq_len * d_model)
    pos1 = pos_table.reshape(-1)
    out = _sc_add(x1, pos1)
    return out.reshape(batch, seq_len, d_model)


# speedup vs baseline: 3.1364x; 3.1364x over previous
import jax
import jax.numpy as jnp
from jax import lax
from jax.experimental import pallas as pl
from jax.experimental.pallas import tpu as pltpu
from jax.experimental.pallas import tpu_sc as plsc

NC = 2
NS = 16
NW = NC * NS
LANES = 16

BATCH = 4
SEQ_LEN = 4096
D_MODEL = 2048
ROWS_PER_W = SEQ_LEN // NW
TILE_R = 4
N_CHUNKS = ROWS_PER_W // TILE_R


def _sc_body(x_hbm, pos_hbm, out_hbm, *scratch):
    pos_v = scratch[0:2]
    x_v = (scratch[2:6], scratch[6:10])
    ld = scratch[10:12]
    st = scratch[12:14]

    wid = lax.axis_index("s") * NC + lax.axis_index("c")
    s0 = wid * ROWS_PER_W

    def seq_row(k):
        return s0 + k * TILE_R

    def start_loads(k, ring):
        pltpu.async_copy(pos_hbm.at[pl.ds(seq_row(k), TILE_R)], pos_v[ring], ld[ring])
        for b in range(BATCH):
            pltpu.async_copy(
                x_hbm.at[pl.ds(b * SEQ_LEN + seq_row(k), TILE_R)],
                x_v[ring][b], ld[ring],
            )

    def wait_loads(k, ring):
        pltpu.make_async_copy(
            pos_hbm.at[pl.ds(seq_row(k), TILE_R)], pos_v[ring], ld[ring]
        ).wait()
        for b in range(BATCH):
            pltpu.make_async_copy(
                x_hbm.at[pl.ds(b * SEQ_LEN + seq_row(k), TILE_R)],
                x_v[ring][b], ld[ring],
            ).wait()

    def start_stores(k, ring):
        for b in range(BATCH):
            pltpu.async_copy(
                x_v[ring][b],
                out_hbm.at[pl.ds(b * SEQ_LEN + seq_row(k), TILE_R)], st[ring],
            )

    def wait_stores(k, ring):
        for b in range(BATCH):
            pltpu.make_async_copy(
                x_v[ring][b],
                out_hbm.at[pl.ds(b * SEQ_LEN + seq_row(k), TILE_R)], st[ring],
            ).wait()

    def compute(ring):
        bufs = x_v[ring]
        pv = pos_v[ring]

        def row_body(r, _):
            @plsc.parallel_loop(0, D_MODEL, step=LANES, unroll=16)
            def _(j):
                p = pv[r, pl.ds(j, LANES)]
                for b in range(BATCH):
                    bufs[b][r, pl.ds(j, LANES)] = bufs[b][r, pl.ds(j, LANES)] + p

            return 0

        lax.fori_loop(0, TILE_R, row_body, 0)

    start_loads(0, 0)
    start_loads(1, 1)

    def pair_body(p, _):
        k0 = p * 2
        for ring in range(2):
            k = k0 + ring
            wait_loads(k, ring)
            compute(ring)
            start_stores(k, ring)

        for ring in range(2):
            k = k0 + ring

            @pl.when(k + 2 < N_CHUNKS)
            def _():
                wait_stores(k, ring)
                start_loads(k + 2, ring)

        return 0

    lax.fori_loop(0, N_CHUNKS // 2, pair_body, 0)

    wait_stores(N_CHUNKS - 2, 0)
    wait_stores(N_CHUNKS - 1, 1)


def _sc_add(x2, pos_table):
    k = pl.kernel(
        _sc_body,
        out_type=jax.ShapeDtypeStruct((BATCH * SEQ_LEN, D_MODEL), jnp.float32),
        mesh=plsc.VectorSubcoreMesh(core_axis_name="c", subcore_axis_name="s"),
        scratch_types=(
            [pltpu.VMEM((TILE_R, D_MODEL), jnp.float32) for _ in range(10)]
            + [pltpu.SemaphoreType.DMA for _ in range(4)]
        ),
    )
    return k(x2, pos_table)


def kernel(x, pos_table):
    batch, seq_len, d_model = x.shape
    x2 = x.reshape(batch * seq_len, d_model)
    out = _sc_add(x2, pos_table)
    return out.reshape(batch, seq_len, d_model)


# speedup vs baseline: 3.1610x; 1.0079x over previous
import jax
import jax.numpy as jnp
from jax import lax
from jax.experimental import pallas as pl
from jax.experimental.pallas import tpu as pltpu
from jax.experimental.pallas import tpu_sc as plsc

NC = 2
NS = 16
NW = NC * NS
LANES = 16

BATCH = 4
SEQ_LEN = 4096
D_MODEL = 2048
ROWS_PER_W = SEQ_LEN // NW
TILE_R = 4
N_CHUNKS = ROWS_PER_W // TILE_R


def _sc_body(x_hbm, pos_hbm, out_hbm, *scratch):
    pos_v = scratch[0:2]
    x_v = scratch[2:4]
    ld = scratch[4:6]
    st = scratch[6:8]

    wid = lax.axis_index("s") * NC + lax.axis_index("c")
    s0 = wid * ROWS_PER_W

    def seq_row(k):
        return s0 + k * TILE_R

    def start_loads(k, ring):
        r = seq_row(k)
        pltpu.async_copy(pos_hbm.at[pl.ds(r, TILE_R)], pos_v[ring], ld[ring])
        pltpu.async_copy(x_hbm.at[:, pl.ds(r, TILE_R), :], x_v[ring], ld[ring])

    def wait_loads(k, ring):
        r = seq_row(k)
        pltpu.make_async_copy(
            pos_hbm.at[pl.ds(r, TILE_R)], pos_v[ring], ld[ring]
        ).wait()
        pltpu.make_async_copy(
            x_hbm.at[:, pl.ds(r, TILE_R), :], x_v[ring], ld[ring]
        ).wait()

    def start_stores(k, ring):
        r = seq_row(k)
        pltpu.async_copy(x_v[ring], out_hbm.at[:, pl.ds(r, TILE_R), :], st[ring])

    def wait_stores(k, ring):
        r = seq_row(k)
        pltpu.make_async_copy(
            x_v[ring], out_hbm.at[:, pl.ds(r, TILE_R), :], st[ring]
        ).wait()

    def compute(ring):
        buf = x_v[ring]
        pv = pos_v[ring]

        def row_body(r, _):
            @plsc.parallel_loop(0, D_MODEL, step=LANES, unroll=16)
            def _(j):
                p = pv[r, pl.ds(j, LANES)]
                for b in range(BATCH):
                    buf[b, r, pl.ds(j, LANES)] = buf[b, r, pl.ds(j, LANES)] + p

            return 0

        lax.fori_loop(0, TILE_R, row_body, 0)

    start_loads(0, 0)
    start_loads(1, 1)

    def pair_body(p, _):
        k0 = p * 2
        for ring in range(2):
            k = k0 + ring
            wait_loads(k, ring)
            compute(ring)
            start_stores(k, ring)

        for ring in range(2):
            k = k0 + ring

            @pl.when(k + 2 < N_CHUNKS)
            def _():
                wait_stores(k, ring)
                start_loads(k + 2, ring)

        return 0

    lax.fori_loop(0, N_CHUNKS // 2, pair_body, 0)

    wait_stores(N_CHUNKS - 2, 0)
    wait_stores(N_CHUNKS - 1, 1)


def _sc_add(x, pos_table):
    k = pl.kernel(
        _sc_body,
        out_type=jax.ShapeDtypeStruct((BATCH, SEQ_LEN, D_MODEL), jnp.float32),
        mesh=plsc.VectorSubcoreMesh(core_axis_name="c", subcore_axis_name="s"),
        scratch_types=(
            [pltpu.VMEM((TILE_R, D_MODEL), jnp.float32) for _ in range(2)]
            + [pltpu.VMEM((BATCH, TILE_R, D_MODEL), jnp.float32) for _ in range(2)]
            + [pltpu.SemaphoreType.DMA for _ in range(4)]
        ),
    )
    return k(x, pos_table)


def kernel(x, pos_table):
    return _sc_add(x, pos_table)
